# baseline (speedup 1.0000x reference)
import functools

import jax
import jax.numpy as jnp
from jax import lax
from jax.experimental import pallas as pl
from jax.experimental.pallas import tpu as pltpu
from jax.experimental.pallas import tpu_sc as plsc

BATCH = 4096
SEQ = 200
EMBED = 32

NC = 2
NS = 16
NW = NC * NS
BPW = BATCH // NW
IDX_PER_W = BPW * SEQ
ROWS_PER_CHUNK = 4
CHUNK_LEN = ROWS_PER_CHUNK * SEQ
CHUNKS = BPW // ROWS_PER_CHUNK

_mesh = plsc.VectorSubcoreMesh(
    core_axis_name="c", subcore_axis_name="s", num_cores=NC, num_subcores=NS
)


@functools.partial(
    pl.kernel,
    out_type=jax.ShapeDtypeStruct((NW, BPW * EMBED), jnp.float32),
    mesh=_mesh,
    scratch_types=[
        pltpu.VMEM((CHUNKS, CHUNK_LEN), jnp.int32),
        pltpu.VMEM((CHUNK_LEN, EMBED), jnp.bfloat16),
        pltpu.VMEM((CHUNK_LEN, EMBED), jnp.bfloat16),
        pltpu.VMEM((BPW * EMBED,), jnp.float32),
        pltpu.SemaphoreType.DMA,
        pltpu.SemaphoreType.DMA,
    ],
    compiler_params=pltpu.CompilerParams(
        use_tc_tiling_on_sc=False, needs_layout_passes=False
    ),
)
def _sc_embed(x_hbm, table_hbm, out_hbm, idx_v, buf0, buf1, out_v, sem0, sem1):
    wid = lax.axis_index("c") * NS + lax.axis_index("s")
    bufs = (buf0, buf1)
    sems = (sem0, sem1)

    pltpu.sync_copy(x_hbm.at[pl.ds(wid * CHUNKS, CHUNKS)], idx_v)

    def fire(b, g):
        for k in range(CHUNK_LEN // 16):
            ivec = idx_v[g, pl.ds(k * 16, 16)]
            pltpu.async_copy(
                table_hbm.at[ivec],
                bufs[b].at[pl.ds(k * 16, 16)],
                sems[b],
            )

    def drain(b):
        for k in range(CHUNK_LEN // 16):
            pltpu.make_async_copy(
                table_hbm.at[idx_v[0, pl.ds(k * 16, 16)]],
                bufs[b].at[pl.ds(k * 16, 16)],
                sems[b],
            ).wait()

    lanes2 = lax.iota(jnp.int32, 16) * 2
    inv = jnp.full((16,), 1.0 / SEQ, jnp.float32)
    himask = jnp.int32(-65536)

    def reduce_chunk(b, g):
        buf = bufs[b]
        for c in range(ROWS_PER_CHUNK):
            base = c * SEQ

            def rbody(r, accs):
                row = base + r * 4
                new = list(accs)
                for i in range(4):
                    w = plsc.bitcast(buf[row + i, :], jnp.int32)
                    ev = plsc.bitcast(w << 16, jnp.float32)
                    od = plsc.bitcast(w & himask, jnp.float32)
                    new[2 * i] = new[2 * i] + ev
                    new[2 * i + 1] = new[2 * i + 1] + od
                return tuple(new)

            zeros = tuple(jnp.zeros((16,), jnp.float32) for _ in range(8))
            accs = lax.fori_loop(0, SEQ // 4, rbody, zeros)
            acc_ev = (accs[0] + accs[2]) + (accs[4] + accs[6])
            acc_od = (accs[1] + accs[3]) + (accs[5] + accs[7])
            obase = (g * ROWS_PER_CHUNK + c) * EMBED
            plsc.store_scatter(out_v, [obase + lanes2], acc_ev * inv)
            plsc.store_scatter(out_v, [obase + lanes2 + 1], acc_od * inv)

    fire(0, 0)

    def outer(o, carry):
        g0 = o * 2
        drain(0)
        fire(1, g0 + 1)
        reduce_chunk(0, g0)
        drain(1)

        @pl.when(o < CHUNKS // 2 - 1)
        def _():
            fire(0, g0 + 2)

        reduce_chunk(1, g0 + 1)
        return carry

    lax.fori_loop(0, CHUNKS // 2, outer, 0)

    pltpu.sync_copy(out_v, out_hbm.at[wid])


def kernel(x, table):
    x2 = x.reshape(-1, CHUNK_LEN).astype(jnp.int32)
    tb = table.astype(jnp.bfloat16)
    return _sc_embed(x2, tb).reshape(BATCH, EMBED)


# speedup vs baseline: 1.1733x; 1.1733x over previous
import functools

import jax
import jax.numpy as jnp
from jax import lax
from jax.experimental import pallas as pl
from jax.experimental.pallas import tpu as pltpu
from jax.experimental.pallas import tpu_sc as plsc

BATCH = 4096
SEQ = 200
EMBED = 32

NC = 2
NS = 16
NW = NC * NS
BPW = BATCH // NW
IDX_PER_W = BPW * SEQ
ROWS_PER_CHUNK = 4
CHUNK_LEN = ROWS_PER_CHUNK * SEQ
CHUNKS = BPW // ROWS_PER_CHUNK

_mesh = plsc.VectorSubcoreMesh(
    core_axis_name="c", subcore_axis_name="s", num_cores=NC, num_subcores=NS
)


@functools.partial(
    pl.kernel,
    out_type=jax.ShapeDtypeStruct((NW, BPW * EMBED), jnp.float32),
    mesh=_mesh,
    scratch_types=[
        pltpu.VMEM((CHUNKS, CHUNK_LEN), jnp.int32),
        pltpu.VMEM((CHUNK_LEN, EMBED), jnp.float32),
        pltpu.VMEM((CHUNK_LEN, EMBED), jnp.float32),
        pltpu.VMEM((BPW * EMBED,), jnp.float32),
        pltpu.SemaphoreType.DMA,
        pltpu.SemaphoreType.DMA,
    ],
    compiler_params=pltpu.CompilerParams(use_tc_tiling_on_sc=False),
)
def _sc_embed(x_hbm, table_hbm, out_hbm, idx_v, buf0, buf1, out_v, sem0, sem1):
    wid = lax.axis_index("c") * NS + lax.axis_index("s")
    bufs = (buf0, buf1)
    sems = (sem0, sem1)

    pltpu.sync_copy(x_hbm.at[pl.ds(wid * CHUNKS, CHUNKS)], idx_v)

    def fire(b, g):
        pltpu.async_copy(table_hbm.at[idx_v.at[g]], bufs[b], sems[b])

    def drain(b):
        pltpu.make_async_copy(
            table_hbm.at[idx_v.at[0]], bufs[b], sems[b]
        ).wait()

    inv = jnp.full((16,), 1.0 / SEQ, jnp.float32)

    def reduce_chunk(b, g):
        buf = bufs[b]
        for c in range(ROWS_PER_CHUNK):
            base = c * SEQ

            def rbody(r, accs):
                row = base + r * 4
                new = []
                for i in range(4):
                    for h in range(2):
                        v = buf[row + i, pl.ds(h * 16, 16)]
                        new.append(accs[i * 2 + h] + v)
                return tuple(new)

            zeros = tuple(jnp.zeros((16,), jnp.float32) for _ in range(8))
            accs = lax.fori_loop(0, SEQ // 4, rbody, zeros)
            half0 = (accs[0] + accs[2]) + (accs[4] + accs[6])
            half1 = (accs[1] + accs[3]) + (accs[5] + accs[7])
            obase = (g * ROWS_PER_CHUNK + c) * EMBED
            out_v[pl.ds(obase, 16)] = half0 * inv
            out_v[pl.ds(obase + 16, 16)] = half1 * inv

    fire(0, 0)

    def outer(o, carry):
        g0 = o * 2
        drain(0)
        fire(1, g0 + 1)
        reduce_chunk(0, g0)
        drain(1)

        @pl.when(o < CHUNKS // 2 - 1)
        def _():
            fire(0, g0 + 2)

        reduce_chunk(1, g0 + 1)
        return carry

    lax.fori_loop(0, CHUNKS // 2, outer, 0)

    pltpu.sync_copy(out_v, out_hbm.at[wid])


def kernel(x, table):
    x2 = x.reshape(-1, CHUNK_LEN).astype(jnp.int32)
    return _sc_embed(x2, table).reshape(BATCH, EMBED)
